# baseline (speedup 1.0000x reference)
import jax
import jax.numpy as jnp
from jax.experimental import pallas as pl
from jax.experimental.pallas import tpu as pltpu

N = 100000
H = 128
HID = 2 * H
C = 128
G3 = 3 * HID

BLOCK = 4000


def _drnn_block_kernel(xa_ref, xf_ref, ph_ref, sh_ref, enc_ref,
                       wd_ih_ref, wd_hh_ref, ww_ih_ref, ww_hh_ref,
                       bd_ih_ref, bd_hh_ref, bw_ih_ref, bw_hh_ref,
                       w_h_ref, b_h_ref, w_pa_ref, w_pf_ref, b_p_ref,
                       h_out_ref, probs_out_ref):
    f32 = jnp.float32
    bf16 = jnp.bfloat16
    enc = enc_ref[...]

    def gru(x, h, wi_t, wh_t, bi, bh):
        gi = jnp.dot(x, wi_t, preferred_element_type=f32) + bi
        gh = jnp.dot(h, wh_t, preferred_element_type=f32) + bh
        r = jax.nn.sigmoid(gi[:, :HID] + gh[:, :HID])
        z = jax.nn.sigmoid(gi[:, HID:2 * HID] + gh[:, HID:2 * HID])
        n = jnp.tanh(gi[:, 2 * HID:] + r * gh[:, 2 * HID:])
        return (1.0 - z) * n + z * h

    ha = gru(xa_ref[...], jnp.concatenate([ph_ref[...], enc], axis=1),
             wd_ih_ref[...], wd_hh_ref[...], bd_ih_ref[...], bd_hh_ref[...])
    hf = gru(xf_ref[...], jnp.concatenate([sh_ref[...], enc], axis=1),
             ww_ih_ref[...], ww_hh_ref[...], bw_ih_ref[...], bw_hh_ref[...])

    hcat = jnp.concatenate([ha, hf], axis=1)
    h_out_ref[...] = jnp.tanh(
        jnp.dot(hcat, w_h_ref[...], preferred_element_type=f32) + b_h_ref[...])

    pa = jnp.sum(ha * w_pa_ref[...], axis=1, keepdims=True)
    pf = jnp.sum(hf * w_pf_ref[...], axis=1, keepdims=True)
    probs_out_ref[...] = jax.nn.sigmoid(
        jnp.concatenate([pa, pf], axis=1) + b_p_ref[...])


def kernel(parent_output_label, sibling_output_label, parent_h, sibling_h, encoding,
           d_W_ih, d_W_hh, d_b_ih, d_b_hh,
           w_W_ih, w_W_hh, w_b_ih, w_b_hh,
           W_pa, b_pa, W_pf, b_pf, W_ha, b_ha, W_hf, b_hf):
    wd_ih_t = d_W_ih.T
    wd_hh_t = d_W_hh.T
    ww_ih_t = w_W_ih.T
    ww_hh_t = w_W_hh.T
    w_h = jnp.concatenate([W_ha.T, W_hf.T], axis=0)
    b_h = (b_ha + b_hf).reshape(1, H)
    b_p = jnp.concatenate([b_pa, b_pf]).reshape(1, 2)

    row = lambda i: (i, 0)
    fixed = lambda i: (0, 0)
    act_spec = pl.BlockSpec((BLOCK, H), row)
    grid = N // BLOCK

    h_out, probs = pl.pallas_call(
        _drnn_block_kernel,
        grid=(grid,),
        in_specs=[
            act_spec, act_spec, act_spec, act_spec, act_spec,
            pl.BlockSpec((C, G3), fixed),
            pl.BlockSpec((HID, G3), fixed),
            pl.BlockSpec((C, G3), fixed),
            pl.BlockSpec((HID, G3), fixed),
            pl.BlockSpec((1, G3), fixed),
            pl.BlockSpec((1, G3), fixed),
            pl.BlockSpec((1, G3), fixed),
            pl.BlockSpec((1, G3), fixed),
            pl.BlockSpec((2 * HID, H), fixed),
            pl.BlockSpec((1, H), fixed),
            pl.BlockSpec((1, HID), fixed),
            pl.BlockSpec((1, HID), fixed),
            pl.BlockSpec((1, 2), fixed),
        ],
        out_specs=[
            pl.BlockSpec((BLOCK, H), row),
            pl.BlockSpec((BLOCK, 2), row),
        ],
        out_shape=[
            jax.ShapeDtypeStruct((N, H), jnp.float32),
            jax.ShapeDtypeStruct((N, 2), jnp.float32),
        ],
        compiler_params=pltpu.CompilerParams(
            dimension_semantics=("arbitrary",),
        ),
    )(parent_output_label, sibling_output_label, parent_h, sibling_h, encoding,
      wd_ih_t, wd_hh_t, ww_ih_t, ww_hh_t,
      d_b_ih.reshape(1, G3), d_b_hh.reshape(1, G3),
      w_b_ih.reshape(1, G3), w_b_hh.reshape(1, G3),
      w_h, b_h, W_pa, W_pf, b_p)
    return (h_out, probs)


# speedup vs baseline: 1.0067x; 1.0067x over previous
import jax
import jax.numpy as jnp
from jax.experimental import pallas as pl
from jax.experimental.pallas import tpu as pltpu

N = 100000
H = 128
HID = 2 * H
C = 128

BLOCK = 2000


def _drnn_block_kernel(xa_ref, xf_ref, ph_ref, sh_ref, enc_ref,
                       wrz_d_ref, win_d_ref, whn_d_ref,
                       wrz_w_ref, win_w_ref, whn_w_ref,
                       brz_d_ref, bin_d_ref, bhn_d_ref,
                       brz_w_ref, bin_w_ref, bhn_w_ref,
                       wha_ref, whf_ref, b_h_ref, w_pa_ref, w_pf_ref, b_p_ref,
                       h_out_ref, probs_out_ref):
    f32 = jnp.float32
    enc = enc_ref[...]

    def gru(x, p, wrz, win, whn, brz, bin_, bhn):
        hp = jnp.concatenate([p, enc], axis=1)
        xcat = jnp.concatenate([x, hp], axis=1)
        trz = jnp.tanh(jnp.dot(xcat, wrz, preferred_element_type=f32) + brz)
        i_n = jnp.dot(x, win, preferred_element_type=f32) + bin_
        h_n2 = jnp.dot(hp, whn, preferred_element_type=f32) + bhn
        n = jnp.tanh(i_n + h_n2 * (trz[:, :HID] + 1.0))
        z = 0.5 * trz[:, HID:] + 0.5
        return n + z * (hp - n)

    ha = gru(xa_ref[...], ph_ref[...], wrz_d_ref[...], win_d_ref[...],
             whn_d_ref[...], brz_d_ref[...], bin_d_ref[...], bhn_d_ref[...])
    hf = gru(xf_ref[...], sh_ref[...], wrz_w_ref[...], win_w_ref[...],
             whn_w_ref[...], brz_w_ref[...], bin_w_ref[...], bhn_w_ref[...])

    h_out_ref[...] = jnp.tanh(
        jnp.dot(ha, wha_ref[...], preferred_element_type=f32)
        + jnp.dot(hf, whf_ref[...], preferred_element_type=f32)
        + b_h_ref[...])

    pa = jnp.sum(ha * w_pa_ref[...], axis=1, keepdims=True)
    pf = jnp.sum(hf * w_pf_ref[...], axis=1, keepdims=True)
    probs_out_ref[...] = jax.nn.sigmoid(
        jnp.concatenate([pa, pf], axis=1) + b_p_ref[...])


def _prep(W_ih, W_hh, b_ih, b_hh):
    wi, wh = W_ih.T, W_hh.T
    wrz = 0.5 * jnp.concatenate([wi[:, :2 * HID], wh[:, :2 * HID]], axis=0)
    brz = (0.5 * (b_ih[:2 * HID] + b_hh[:2 * HID])).reshape(1, 2 * HID)
    win = wi[:, 2 * HID:]
    bin_ = b_ih[2 * HID:].reshape(1, HID)
    whn = 0.5 * wh[:, 2 * HID:]
    bhn = (0.5 * b_hh[2 * HID:]).reshape(1, HID)
    return wrz, win, whn, brz, bin_, bhn


def kernel(parent_output_label, sibling_output_label, parent_h, sibling_h, encoding,
           d_W_ih, d_W_hh, d_b_ih, d_b_hh,
           w_W_ih, w_W_hh, w_b_ih, w_b_hh,
           W_pa, b_pa, W_pf, b_pf, W_ha, b_ha, W_hf, b_hf):
    wrz_d, win_d, whn_d, brz_d, bin_d, bhn_d = _prep(d_W_ih, d_W_hh, d_b_ih, d_b_hh)
    wrz_w, win_w, whn_w, brz_w, bin_w, bhn_w = _prep(w_W_ih, w_W_hh, w_b_ih, w_b_hh)
    b_h = (b_ha + b_hf).reshape(1, H)
    b_p = jnp.concatenate([b_pa, b_pf]).reshape(1, 2)

    row = lambda i: (i, 0)
    fixed = lambda i: (0, 0)
    act_spec = pl.BlockSpec((BLOCK, H), row)
    wspec = lambda a, b: pl.BlockSpec((a, b), fixed)
    grid = N // BLOCK

    h_out, probs = pl.pallas_call(
        _drnn_block_kernel,
        grid=(grid,),
        in_specs=[
            act_spec, act_spec, act_spec, act_spec, act_spec,
            wspec(3 * H, 2 * HID), wspec(C, HID), wspec(HID, HID),
            wspec(3 * H, 2 * HID), wspec(C, HID), wspec(HID, HID),
            wspec(1, 2 * HID), wspec(1, HID), wspec(1, HID),
            wspec(1, 2 * HID), wspec(1, HID), wspec(1, HID),
            wspec(HID, H), wspec(HID, H), wspec(1, H),
            wspec(1, HID), wspec(1, HID), wspec(1, 2),
        ],
        out_specs=[
            pl.BlockSpec((BLOCK, H), row),
            pl.BlockSpec((BLOCK, 2), row),
        ],
        out_shape=[
            jax.ShapeDtypeStruct((N, H), jnp.float32),
            jax.ShapeDtypeStruct((N, 2), jnp.float32),
        ],
        compiler_params=pltpu.CompilerParams(
            dimension_semantics=("arbitrary",),
        ),
    )(parent_output_label, sibling_output_label, parent_h, sibling_h, encoding,
      wrz_d, win_d, whn_d,
      wrz_w, win_w, whn_w,
      brz_d, bin_d, bhn_d,
      brz_w, bin_w, bhn_w,
      W_ha.T, W_hf.T, b_h, W_pa, W_pf, b_p)
    return (h_out, probs)


# speedup vs baseline: 1.0141x; 1.0074x over previous
import jax
import jax.numpy as jnp
from jax.experimental import pallas as pl
from jax.experimental.pallas import tpu as pltpu

N = 100000
H = 128
HID = 2 * H
C = 128

BLOCK = 2000


def _drnn_block_kernel(xa_ref, xf_ref, ph_ref, sh_ref, enc_ref,
                       wrz_d_ref, win_d_ref, whn_d_ref,
                       wrz_w_ref, win_w_ref, whn_w_ref,
                       brz_d_ref, bin_d_ref, bhn_d_ref,
                       brz_w_ref, bin_w_ref, bhn_w_ref,
                       wha_ref, whf_ref, b_h_ref, w_pa_ref, w_pf_ref, b_p_ref,
                       h_out_ref, probs_out_ref):
    f32 = jnp.float32
    bf16 = jnp.bfloat16
    enc = enc_ref[...]

    def gru(x, p, wrz, win, whn, brz, bin_, bhn):
        hp = jnp.concatenate([p, enc], axis=1).astype(bf16)
        xb = x.astype(bf16)
        xcat = jnp.concatenate([xb, hp], axis=1)
        trz = jnp.tanh(
            jnp.dot(xcat, wrz, preferred_element_type=f32).astype(bf16) + brz)
        i_n = jnp.dot(xb, win, preferred_element_type=f32).astype(bf16) + bin_
        h_n2 = jnp.dot(hp, whn, preferred_element_type=f32).astype(bf16) + bhn
        n = jnp.tanh(i_n + h_n2 * (trz[:, :HID] + 1.0))
        z = 0.5 * trz[:, HID:] + 0.5
        return n + z * (hp - n)

    ha = gru(xa_ref[...], ph_ref[...], wrz_d_ref[...], win_d_ref[...],
             whn_d_ref[...], brz_d_ref[...], bin_d_ref[...], bhn_d_ref[...])
    hf = gru(xf_ref[...], sh_ref[...], wrz_w_ref[...], win_w_ref[...],
             whn_w_ref[...], brz_w_ref[...], bin_w_ref[...], bhn_w_ref[...])

    h_out_ref[...] = jnp.tanh(
        jnp.dot(ha, wha_ref[...], preferred_element_type=f32)
        + jnp.dot(hf, whf_ref[...], preferred_element_type=f32)
        + b_h_ref[...])

    pa = jnp.sum((ha * w_pa_ref[...]).astype(f32), axis=1, keepdims=True)
    pf = jnp.sum((hf * w_pf_ref[...]).astype(f32), axis=1, keepdims=True)
    probs_out_ref[...] = jax.nn.sigmoid(
        jnp.concatenate([pa, pf], axis=1) + b_p_ref[...])


def _prep(W_ih, W_hh, b_ih, b_hh):
    bf16 = jnp.bfloat16
    wi, wh = W_ih.T, W_hh.T
    wrz = (0.5 * jnp.concatenate([wi[:, :2 * HID], wh[:, :2 * HID]], axis=0)).astype(bf16)
    brz = (0.5 * (b_ih[:2 * HID] + b_hh[:2 * HID])).reshape(1, 2 * HID).astype(bf16)
    win = wi[:, 2 * HID:].astype(bf16)
    bin_ = b_ih[2 * HID:].reshape(1, HID).astype(bf16)
    whn = (0.5 * wh[:, 2 * HID:]).astype(bf16)
    bhn = (0.5 * b_hh[2 * HID:]).reshape(1, HID).astype(bf16)
    return wrz, win, whn, brz, bin_, bhn


def kernel(parent_output_label, sibling_output_label, parent_h, sibling_h, encoding,
           d_W_ih, d_W_hh, d_b_ih, d_b_hh,
           w_W_ih, w_W_hh, w_b_ih, w_b_hh,
           W_pa, b_pa, W_pf, b_pf, W_ha, b_ha, W_hf, b_hf):
    bf16 = jnp.bfloat16
    wrz_d, win_d, whn_d, brz_d, bin_d, bhn_d = _prep(d_W_ih, d_W_hh, d_b_ih, d_b_hh)
    wrz_w, win_w, whn_w, brz_w, bin_w, bhn_w = _prep(w_W_ih, w_W_hh, w_b_ih, w_b_hh)
    b_h = (b_ha + b_hf).reshape(1, H)
    b_p = jnp.concatenate([b_pa, b_pf]).reshape(1, 2)

    row = lambda i: (i, 0)
    fixed = lambda i: (0, 0)
    act_spec = pl.BlockSpec((BLOCK, H), row)
    wspec = lambda a, b: pl.BlockSpec((a, b), fixed)
    grid = N // BLOCK

    h_out, probs = pl.pallas_call(
        _drnn_block_kernel,
        grid=(grid,),
        in_specs=[
            act_spec, act_spec, act_spec, act_spec, act_spec,
            wspec(3 * H, 2 * HID), wspec(C, HID), wspec(HID, HID),
            wspec(3 * H, 2 * HID), wspec(C, HID), wspec(HID, HID),
            wspec(1, 2 * HID), wspec(1, HID), wspec(1, HID),
            wspec(1, 2 * HID), wspec(1, HID), wspec(1, HID),
            wspec(HID, H), wspec(HID, H), wspec(1, H),
            wspec(1, HID), wspec(1, HID), wspec(1, 2),
        ],
        out_specs=[
            pl.BlockSpec((BLOCK, H), row),
            pl.BlockSpec((BLOCK, 2), row),
        ],
        out_shape=[
            jax.ShapeDtypeStruct((N, H), jnp.float32),
            jax.ShapeDtypeStruct((N, 2), jnp.float32),
        ],
        compiler_params=pltpu.CompilerParams(
            dimension_semantics=("arbitrary",),
        ),
    )(parent_output_label, sibling_output_label, parent_h, sibling_h, encoding,
      wrz_d, win_d, whn_d,
      wrz_w, win_w, whn_w,
      brz_d, bin_d, bhn_d,
      brz_w, bin_w, bhn_w,
      W_ha.T.astype(bf16), W_hf.T.astype(bf16), b_h,
      W_pa.astype(bf16), W_pf.astype(bf16), b_p)
    return (h_out, probs)


# speedup vs baseline: 1.0177x; 1.0036x over previous
import jax
import jax.numpy as jnp
from jax.experimental import pallas as pl
from jax.experimental.pallas import tpu as pltpu

N = 100000
H = 128
HID = 2 * H
C = 128

BLOCK = 2000
SUB = 1000


def _drnn_block_kernel(xa_ref, xf_ref, ph_ref, sh_ref, enc_ref,
                       wrz_d_ref, win_d_ref, whn_d_ref,
                       wrz_w_ref, win_w_ref, whn_w_ref,
                       brz_d_ref, bin_d_ref, bhn_d_ref,
                       brz_w_ref, bin_w_ref, bhn_w_ref,
                       wha_ref, whf_ref, b_h_ref, w_pa_ref, w_pf_ref, b_p_ref,
                       h_out_ref, probs_out_ref):
    f32 = jnp.float32
    bf16 = jnp.bfloat16

    def gru(x, p, enc, wrz, win, whn, brz, bin_, bhn):
        hp = jnp.concatenate([p, enc], axis=1).astype(bf16)
        xb = x.astype(bf16)
        xcat = jnp.concatenate([xb, hp], axis=1)
        trz = jnp.tanh(
            jnp.dot(xcat, wrz, preferred_element_type=f32).astype(bf16) + brz)
        i_n = jnp.dot(xb, win, preferred_element_type=f32).astype(bf16) + bin_
        h_n2 = jnp.dot(hp, whn, preferred_element_type=f32).astype(bf16) + bhn
        n = jnp.tanh(i_n + h_n2 * (trz[:, :HID] + 1.0))
        z = 0.5 * trz[:, HID:] + 0.5
        return n + z * (hp - n)

    for c in range(BLOCK // SUB):
        sl = pl.ds(c * SUB, SUB)
        enc = enc_ref[sl, :]
        ha = gru(xa_ref[sl, :], ph_ref[sl, :], enc,
                 wrz_d_ref[...], win_d_ref[...], whn_d_ref[...],
                 brz_d_ref[...], bin_d_ref[...], bhn_d_ref[...])
        hf = gru(xf_ref[sl, :], sh_ref[sl, :], enc,
                 wrz_w_ref[...], win_w_ref[...], whn_w_ref[...],
                 brz_w_ref[...], bin_w_ref[...], bhn_w_ref[...])

        h_out_ref[sl, :] = jnp.tanh(
            jnp.dot(ha, wha_ref[...], preferred_element_type=f32)
            + jnp.dot(hf, whf_ref[...], preferred_element_type=f32)
            + b_h_ref[...])

        pa = jnp.sum((ha * w_pa_ref[...]).astype(f32), axis=1, keepdims=True)
        pf = jnp.sum((hf * w_pf_ref[...]).astype(f32), axis=1, keepdims=True)
        probs_out_ref[sl, :] = jax.nn.sigmoid(
            jnp.concatenate([pa, pf], axis=1) + b_p_ref[...])


def _prep(W_ih, W_hh, b_ih, b_hh):
    bf16 = jnp.bfloat16
    wi, wh = W_ih.T, W_hh.T
    wrz = (0.5 * jnp.concatenate([wi[:, :2 * HID], wh[:, :2 * HID]], axis=0)).astype(bf16)
    brz = (0.5 * (b_ih[:2 * HID] + b_hh[:2 * HID])).reshape(1, 2 * HID).astype(bf16)
    win = wi[:, 2 * HID:].astype(bf16)
    bin_ = b_ih[2 * HID:].reshape(1, HID).astype(bf16)
    whn = (0.5 * wh[:, 2 * HID:]).astype(bf16)
    bhn = (0.5 * b_hh[2 * HID:]).reshape(1, HID).astype(bf16)
    return wrz, win, whn, brz, bin_, bhn


def kernel(parent_output_label, sibling_output_label, parent_h, sibling_h, encoding,
           d_W_ih, d_W_hh, d_b_ih, d_b_hh,
           w_W_ih, w_W_hh, w_b_ih, w_b_hh,
           W_pa, b_pa, W_pf, b_pf, W_ha, b_ha, W_hf, b_hf):
    bf16 = jnp.bfloat16
    wrz_d, win_d, whn_d, brz_d, bin_d, bhn_d = _prep(d_W_ih, d_W_hh, d_b_ih, d_b_hh)
    wrz_w, win_w, whn_w, brz_w, bin_w, bhn_w = _prep(w_W_ih, w_W_hh, w_b_ih, w_b_hh)
    b_h = (b_ha + b_hf).reshape(1, H)
    b_p = jnp.concatenate([b_pa, b_pf]).reshape(1, 2)

    row = lambda i: (i, 0)
    fixed = lambda i: (0, 0)
    act_spec = pl.BlockSpec((BLOCK, H), row)
    wspec = lambda a, b: pl.BlockSpec((a, b), fixed)
    grid = N // BLOCK

    h_out, probs = pl.pallas_call(
        _drnn_block_kernel,
        grid=(grid,),
        in_specs=[
            act_spec, act_spec, act_spec, act_spec, act_spec,
            wspec(3 * H, 2 * HID), wspec(C, HID), wspec(HID, HID),
            wspec(3 * H, 2 * HID), wspec(C, HID), wspec(HID, HID),
            wspec(1, 2 * HID), wspec(1, HID), wspec(1, HID),
            wspec(1, 2 * HID), wspec(1, HID), wspec(1, HID),
            wspec(HID, H), wspec(HID, H), wspec(1, H),
            wspec(1, HID), wspec(1, HID), wspec(1, 2),
        ],
        out_specs=[
            pl.BlockSpec((BLOCK, H), row),
            pl.BlockSpec((BLOCK, 2), row),
        ],
        out_shape=[
            jax.ShapeDtypeStruct((N, H), jnp.float32),
            jax.ShapeDtypeStruct((N, 2), jnp.float32),
        ],
        compiler_params=pltpu.CompilerParams(
            dimension_semantics=("parallel",),
        ),
    )(parent_output_label, sibling_output_label, parent_h, sibling_h, encoding,
      wrz_d, win_d, whn_d,
      wrz_w, win_w, whn_w,
      brz_d, bin_d, bhn_d,
      brz_w, bin_w, bhn_w,
      W_ha.T.astype(bf16), W_hf.T.astype(bf16), b_h,
      W_pa.astype(bf16), W_pf.astype(bf16), b_p)
    return (h_out, probs)
